# baseline (speedup 1.0000x reference)
import jax
import jax.numpy as jnp
from jax.experimental import pallas as pl
from jax.experimental.pallas import tpu as pltpu

_SEQ_CHUNKS = 8
_VB = 2048


def _make_body(seq_len):
    def _body(x0_ref, x1_ref, w0_ref, w1_ref, b_ref, o_ref, xs_ref):
        i = pl.program_id(0)

        @pl.when(i == 0)
        def _():
            xs_ref[...] = jnp.zeros_like(xs_ref)

        @pl.when(i < _SEQ_CHUNKS)
        def _():
            dh = xs_ref.shape[1] // 2
            xs_ref[:, :dh] += jnp.sum(x0_ref[...], axis=1)
            xs_ref[:, dh:] += jnp.sum(x1_ref[...], axis=1)

        @pl.when(i >= _SEQ_CHUNKS)
        def _():
            dh = xs_ref.shape[1] // 2
            dn = (((1,), (1,)), ((), ()))
            acc = jax.lax.dot_general(
                w0_ref[...], xs_ref[:, :dh], dimension_numbers=dn,
                preferred_element_type=jnp.float32,
            ) + jax.lax.dot_general(
                w1_ref[...], xs_ref[:, dh:], dimension_numbers=dn,
                preferred_element_type=jnp.float32,
            )
            o_ref[...] = acc.T + jnp.float32(seq_len) * b_ref[...]

    return _body


def kernel(input, W, b):
    B, S, D = input.shape
    V = W.shape[0]
    sc = S // _SEQ_CHUNKS
    dh = D // 2
    nvb = -(-V // _VB)

    x_idx = lambda i: jnp.minimum(i, _SEQ_CHUNKS - 1)
    w_idx = lambda i: jnp.maximum(i - _SEQ_CHUNKS, 0)

    out = pl.pallas_call(
        _make_body(S),
        out_shape=jax.ShapeDtypeStruct((B, V), jnp.float32),
        grid=(_SEQ_CHUNKS + nvb,),
        in_specs=[
            pl.BlockSpec((B, sc, dh), lambda i: (0, x_idx(i), 0)),
            pl.BlockSpec((B, sc, dh), lambda i: (0, x_idx(i), 1)),
            pl.BlockSpec((_VB, dh), lambda i: (w_idx(i), 0)),
            pl.BlockSpec((_VB, dh), lambda i: (w_idx(i), 1)),
            pl.BlockSpec((1, _VB), lambda i: (0, w_idx(i))),
        ],
        out_specs=pl.BlockSpec((B, _VB), lambda i: (0, w_idx(i))),
        scratch_shapes=[pltpu.VMEM((B, D), jnp.float32)],
        compiler_params=pltpu.CompilerParams(
            dimension_semantics=("arbitrary",),
            vmem_limit_bytes=50 * 1024 * 1024,
        ),
    )(input, input, W, W, b.reshape(1, V))

    return out


# speedup vs baseline: 1.0007x; 1.0007x over previous
import jax
import jax.numpy as jnp
from jax.experimental import pallas as pl
from jax.experimental.pallas import tpu as pltpu

_SEQ_CHUNKS = 8
_VB = 2048


def _make_body(seq_len):
    def _body(x0_ref, x1_ref, w_ref, b_ref, o_ref, xs_ref):
        i = pl.program_id(0)

        @pl.when(i == 0)
        def _():
            xs_ref[...] = jnp.zeros_like(xs_ref)

        @pl.when(i < _SEQ_CHUNKS)
        def _():
            dh = xs_ref.shape[1] // 2
            xs_ref[:, :dh] += jnp.sum(x0_ref[...], axis=1)
            xs_ref[:, dh:] += jnp.sum(x1_ref[...], axis=1)

        @pl.when(i >= _SEQ_CHUNKS)
        def _():
            acc = jax.lax.dot_general(
                w_ref[...],
                xs_ref[...],
                dimension_numbers=(((1,), (1,)), ((), ())),
                preferred_element_type=jnp.float32,
            )
            o_ref[...] = acc.T + jnp.float32(seq_len) * b_ref[...]

    return _body


def kernel(input, W, b):
    B, S, D = input.shape
    V = W.shape[0]
    sc = S // _SEQ_CHUNKS
    dh = D // 2
    nvb = -(-V // _VB)

    x_idx = lambda i: jnp.minimum(i, _SEQ_CHUNKS - 1)
    w_idx = lambda i: jnp.maximum(i - _SEQ_CHUNKS, 0)

    out = pl.pallas_call(
        _make_body(S),
        out_shape=jax.ShapeDtypeStruct((B, V), jnp.float32),
        grid=(_SEQ_CHUNKS + nvb,),
        in_specs=[
            pl.BlockSpec((B, sc, dh), lambda i: (0, x_idx(i), 0)),
            pl.BlockSpec((B, sc, dh), lambda i: (0, x_idx(i), 1)),
            pl.BlockSpec((_VB, D), lambda i: (w_idx(i), 0)),
            pl.BlockSpec((1, _VB), lambda i: (0, w_idx(i))),
        ],
        out_specs=pl.BlockSpec((B, _VB), lambda i: (0, w_idx(i))),
        scratch_shapes=[pltpu.VMEM((B, D), jnp.float32)],
        compiler_params=pltpu.CompilerParams(
            dimension_semantics=("arbitrary",),
            vmem_limit_bytes=50 * 1024 * 1024,
        ),
    )(input, input, W, b.reshape(1, V))

    return out
